# baseline (speedup 1.0000x reference)
import functools

import jax
import jax.numpy as jnp
import numpy as np
from jax import lax
from jax.experimental import pallas as pl
from jax.experimental.pallas import tpu as pltpu
from jax.experimental.pallas import tpu_sc as plsc

_ALPHA = 4


def kernel(frames):
    B, T, C, H, W = frames.shape
    S = T // _ALPHA

    idx = np.linspace(0.0, T - 1, S).astype(np.int64)
    assert np.array_equal(idx, (np.arange(S) * (T - 1)) // (S - 1))

    NW = 32
    n_slow_planes = B * S * C
    slow_per_w = n_slow_planes // NW
    NBUF = 2

    mesh = plsc.VectorSubcoreMesh(core_axis_name="c", subcore_axis_name="s")

    @functools.partial(
        pl.kernel,
        out_type=jax.ShapeDtypeStruct((B, S, C, H, W), jnp.float32),
        mesh=mesh,
        scratch_types=[
            pltpu.VMEM((H, W), jnp.float32),
            pltpu.VMEM((H, W), jnp.float32),
            pltpu.SemaphoreType.DMA,
            pltpu.SemaphoreType.DMA,
            pltpu.SemaphoreType.DMA,
            pltpu.SemaphoreType.DMA,
        ],
    )
    def slow_gather(src_hbm, slow_hbm, b0, b1, l0, l1, s0, s1):
        buf = (b0, b1)
        lsem = (l0, l1)
        ssem = (s0, s1)
        wid = lax.axis_index("s") * 2 + lax.axis_index("c")

        def unit(i):
            q = wid * slow_per_w + i
            r, c = divmod(q, C)
            b, j = divmod(r, S)
            t = (j * (T - 1)) // (S - 1)
            return src_hbm.at[b, t, c], slow_hbm.at[b, j, c]

        loads = [None] * NBUF
        stores = [None] * NBUF

        def start_load(i):
            src, _ = unit(i)
            loads[i % NBUF] = pltpu.async_copy(src, buf[i % NBUF], lsem[i % NBUF])

        def start_store(i):
            _, dst = unit(i)
            stores[i % NBUF] = pltpu.async_copy(buf[i % NBUF], dst, ssem[i % NBUF])

        start_load(0)
        for i in range(slow_per_w):
            nxt = i + 1
            if nxt < slow_per_w:
                if stores[nxt % NBUF] is not None:
                    stores[nxt % NBUF].wait()
                start_load(nxt)
            loads[i % NBUF].wait()
            start_store(i)
        for i in range(slow_per_w - NBUF, slow_per_w):
            stores[i % NBUF].wait()

    NSPAN = 2
    NSEM = 8

    def copy_body(x_hbm, o_hbm, sems):
        copies = []
        for b in range(B):
            for k in range(NSPAN):
                i = b * NSPAN + k
                sl = pl.ds(k * (T // NSPAN), T // NSPAN)
                copies.append(
                    pltpu.make_async_copy(
                        x_hbm.at[b, sl], o_hbm.at[b, sl], sems.at[i % NSEM]
                    )
                )
        for c in copies:
            c.start()
        for c in copies:
            c.wait()

    fast_copy = pl.pallas_call(
        copy_body,
        in_specs=[pl.BlockSpec(memory_space=pltpu.MemorySpace.HBM)],
        out_specs=pl.BlockSpec(memory_space=pltpu.MemorySpace.HBM),
        out_shape=jax.ShapeDtypeStruct((B, T, C, H, W), jnp.float32),
        scratch_shapes=[pltpu.SemaphoreType.DMA((NSEM,))],
    )

    slow = slow_gather(frames)
    fast = fast_copy(frames)
    return slow, fast


# speedup vs baseline: 31.3531x; 31.3531x over previous
import functools

import jax
import jax.numpy as jnp
import numpy as np
from jax import lax
from jax.experimental import pallas as pl
from jax.experimental.pallas import tpu as pltpu
from jax.experimental.pallas import tpu_sc as plsc

_ALPHA = 4


def kernel(frames):
    B, T, C, H, W = frames.shape
    S = T // _ALPHA

    idx = np.linspace(0.0, T - 1, S).astype(np.int64)
    assert np.array_equal(idx, (np.arange(S) * (T - 1)) // (S - 1))

    NW = 32
    n_fast_planes = B * T * C
    fast_per_w = n_fast_planes // NW
    NBUF = 2

    mesh = plsc.VectorSubcoreMesh(core_axis_name="c", subcore_axis_name="s")

    @functools.partial(
        pl.kernel,
        out_type=jax.ShapeDtypeStruct((B, T, C, H, W), jnp.float32),
        mesh=mesh,
        scratch_types=[
            pltpu.VMEM((H, W), jnp.float32),
            pltpu.VMEM((H, W), jnp.float32),
            pltpu.SemaphoreType.DMA,
            pltpu.SemaphoreType.DMA,
            pltpu.SemaphoreType.DMA,
            pltpu.SemaphoreType.DMA,
        ],
    )
    def fast_copy(src_hbm, fast_hbm, b0, b1, l0, l1, s0, s1):
        buf = (b0, b1)
        lsem = (l0, l1)
        ssem = (s0, s1)
        wid = lax.axis_index("s") * 2 + lax.axis_index("c")

        def unit(i):
            p = wid * fast_per_w + i
            f, c = divmod(p, C)
            b, t = divmod(f, T)
            return src_hbm.at[b, t, c], fast_hbm.at[b, t, c]

        loads = [None] * NBUF
        stores = [None] * NBUF

        def start_load(i):
            src, _ = unit(i)
            loads[i % NBUF] = pltpu.async_copy(src, buf[i % NBUF], lsem[i % NBUF])

        def start_store(i):
            _, dst = unit(i)
            stores[i % NBUF] = pltpu.async_copy(buf[i % NBUF], dst, ssem[i % NBUF])

        start_load(0)
        for i in range(fast_per_w):
            nxt = i + 1
            if nxt < fast_per_w:
                if stores[nxt % NBUF] is not None:
                    stores[nxt % NBUF].wait()
                start_load(nxt)
            loads[i % NBUF].wait()
            start_store(i)
        for i in range(fast_per_w - NBUF, fast_per_w):
            stores[i % NBUF].wait()

    def gather_body(x_ref, o_ref):
        o_ref[...] = x_ref[...]

    slow_gather = pl.pallas_call(
        gather_body,
        grid=(B, S),
        in_specs=[
            pl.BlockSpec(
                (1, 1, C, H, W),
                lambda b, j: (b, (j * (T - 1)) // (S - 1), 0, 0, 0),
            )
        ],
        out_specs=pl.BlockSpec((1, 1, C, H, W), lambda b, j: (b, j, 0, 0, 0)),
        out_shape=jax.ShapeDtypeStruct((B, S, C, H, W), jnp.float32),
    )

    fast = fast_copy(frames)
    slow = slow_gather(frames)
    return slow, fast


# speedup vs baseline: 34.4539x; 1.0989x over previous
import functools

import jax
import jax.numpy as jnp
import numpy as np
from jax import lax
from jax.experimental import pallas as pl
from jax.experimental.pallas import tpu as pltpu
from jax.experimental.pallas import tpu_sc as plsc

_ALPHA = 4


def kernel(frames):
    B, T, C, H, W = frames.shape
    S = T // _ALPHA

    idx = np.linspace(0.0, T - 1, S).astype(np.int64)
    assert np.array_equal(idx, (np.arange(S) * (T - 1)) // (S - 1))

    NW = 32
    n_slow_planes = B * S * C
    slow_per_w = n_slow_planes // NW
    NBUF = 2

    mesh = plsc.VectorSubcoreMesh(core_axis_name="c", subcore_axis_name="s")

    @functools.partial(
        pl.kernel,
        out_type=jax.ShapeDtypeStruct((B, S, C, H, W), jnp.float32),
        mesh=mesh,
        scratch_types=[
            pltpu.VMEM((H, W), jnp.float32),
            pltpu.VMEM((H, W), jnp.float32),
            pltpu.SemaphoreType.DMA,
            pltpu.SemaphoreType.DMA,
            pltpu.SemaphoreType.DMA,
            pltpu.SemaphoreType.DMA,
        ],
    )
    def slow_gather(src_hbm, slow_hbm, b0, b1, l0, l1, s0, s1):
        buf = (b0, b1)
        lsem = (l0, l1)
        ssem = (s0, s1)
        wid = lax.axis_index("s") * 2 + lax.axis_index("c")

        def unit(i):
            q = wid * slow_per_w + i
            r, c = divmod(q, C)
            b, j = divmod(r, S)
            t = (j * (T - 1)) // (S - 1)
            return src_hbm.at[b, t, c], slow_hbm.at[b, j, c]

        loads = [None] * NBUF
        stores = [None] * NBUF

        def start_load(i):
            src, _ = unit(i)
            loads[i % NBUF] = pltpu.async_copy(src, buf[i % NBUF], lsem[i % NBUF])

        def start_store(i):
            _, dst = unit(i)
            stores[i % NBUF] = pltpu.async_copy(buf[i % NBUF], dst, ssem[i % NBUF])

        start_load(0)
        for i in range(slow_per_w):
            nxt = i + 1
            if nxt < slow_per_w:
                if stores[nxt % NBUF] is not None:
                    stores[nxt % NBUF].wait()
                start_load(nxt)
            loads[i % NBUF].wait()
            start_store(i)
        for i in range(slow_per_w - NBUF, slow_per_w):
            stores[i % NBUF].wait()

    FBLK = 8

    def copy_body(x_ref, o_ref):
        o_ref[...] = x_ref[...]

    fast_copy = pl.pallas_call(
        copy_body,
        grid=(B, T // FBLK),
        in_specs=[
            pl.BlockSpec((1, FBLK, C, H, W), lambda i, j: (i, j, 0, 0, 0))
        ],
        out_specs=pl.BlockSpec((1, FBLK, C, H, W), lambda i, j: (i, j, 0, 0, 0)),
        out_shape=jax.ShapeDtypeStruct((B, T, C, H, W), jnp.float32),
    )

    slow = slow_gather(frames)
    fast = fast_copy(frames)
    return slow, fast


# speedup vs baseline: 34.7206x; 1.0077x over previous
import functools

import jax
import jax.numpy as jnp
import numpy as np
from jax import lax
from jax.experimental import pallas as pl
from jax.experimental.pallas import tpu as pltpu
from jax.experimental.pallas import tpu_sc as plsc

_ALPHA = 4


def kernel(frames):
    B, T, C, H, W = frames.shape
    S = T // _ALPHA

    idx = np.linspace(0.0, T - 1, S).astype(np.int64)
    assert np.array_equal(idx, (np.arange(S) * (T - 1)) // (S - 1))

    NW = 32
    n_slow_planes = B * S * C
    slow_per_w = n_slow_planes // NW
    NBUF = 2

    mesh = plsc.VectorSubcoreMesh(core_axis_name="c", subcore_axis_name="s")

    @functools.partial(
        pl.kernel,
        out_type=jax.ShapeDtypeStruct((B, S, C, H, W), jnp.float32),
        mesh=mesh,
        scratch_types=[
            pltpu.VMEM((H, W), jnp.float32),
            pltpu.VMEM((H, W), jnp.float32),
            pltpu.SemaphoreType.DMA,
            pltpu.SemaphoreType.DMA,
            pltpu.SemaphoreType.DMA,
            pltpu.SemaphoreType.DMA,
        ],
    )
    def slow_gather(src_hbm, slow_hbm, b0, b1, l0, l1, s0, s1):
        buf = (b0, b1)
        lsem = (l0, l1)
        ssem = (s0, s1)
        wid = lax.axis_index("s") * 2 + lax.axis_index("c")

        def unit(i):
            q = wid * slow_per_w + i
            r, c = divmod(q, C)
            b, j = divmod(r, S)
            t = (j * (T - 1)) // (S - 1)
            return src_hbm.at[b, t, c], slow_hbm.at[b, j, c]

        loads = [None] * NBUF
        stores = [None] * NBUF

        def start_load(i):
            src, _ = unit(i)
            loads[i % NBUF] = pltpu.async_copy(src, buf[i % NBUF], lsem[i % NBUF])

        def start_store(i):
            _, dst = unit(i)
            stores[i % NBUF] = pltpu.async_copy(buf[i % NBUF], dst, ssem[i % NBUF])

        start_load(0)
        for i in range(slow_per_w):
            nxt = i + 1
            if nxt < slow_per_w:
                if stores[nxt % NBUF] is not None:
                    stores[nxt % NBUF].wait()
                start_load(nxt)
            loads[i % NBUF].wait()
            start_store(i)
        for i in range(slow_per_w - NBUF, slow_per_w):
            stores[i % NBUF].wait()

    FBLK = 16

    def copy_body(x_ref, o_ref):
        o_ref[...] = x_ref[...]

    fast_copy = pl.pallas_call(
        copy_body,
        grid=(B, T // FBLK),
        in_specs=[
            pl.BlockSpec((1, FBLK, C, H, W), lambda i, j: (i, j, 0, 0, 0))
        ],
        out_specs=pl.BlockSpec((1, FBLK, C, H, W), lambda i, j: (i, j, 0, 0, 0)),
        out_shape=jax.ShapeDtypeStruct((B, T, C, H, W), jnp.float32),
    )

    slow = slow_gather(frames)
    fast = fast_copy(frames)
    return slow, fast
